# Initial kernel scaffold; baseline (speedup 1.0000x reference)
import jax
import jax.numpy as jnp
from jax.experimental import pallas as pl


def kernel(x, adj):
    raise NotImplementedError("write your pallas kernel here")


# speedup vs baseline: 1.0147x; 1.0147x over previous
import jax
import jax.numpy as jnp
from jax.experimental import pallas as pl
from jax.experimental.pallas import tpu as pltpu

_BI = 400


def _pool_kernel(x_ref, adj_ref, out_ref):
    i = pl.program_id(0)
    mask = (adj_ref[...] == 1).astype(jnp.bfloat16)
    xb = x_ref[...].astype(jnp.bfloat16)
    acc = jnp.dot(mask, xb, preferred_element_type=jnp.float32)
    out_ref[...] = acc + x_ref[pl.ds(i * _BI, _BI), :]


def kernel(x, adj):
    n, f = x.shape
    grid = (n // _BI,)
    return pl.pallas_call(
        _pool_kernel,
        grid=grid,
        in_specs=[
            pl.BlockSpec((n, f), lambda i: (0, 0)),
            pl.BlockSpec((_BI, n), lambda i: (i, 0)),
        ],
        out_specs=pl.BlockSpec((_BI, f), lambda i: (i, 0)),
        out_shape=jax.ShapeDtypeStruct((n, f), jnp.float32),
        compiler_params=pltpu.CompilerParams(
            dimension_semantics=("parallel",),
        ),
    )(x, adj)


# speedup vs baseline: 1.0170x; 1.0023x over previous
import jax
import jax.numpy as jnp
from jax.experimental import pallas as pl
from jax.experimental.pallas import tpu as pltpu

_BI = 400


def _pool_kernel(x_ref, adj_ref, out_ref):
    i = pl.program_id(0)
    mask = (adj_ref[...] == 1).astype(jnp.bfloat16)
    xb = x_ref[...].astype(jnp.bfloat16)
    acc = jnp.dot(mask, xb, preferred_element_type=jnp.float32)
    out_ref[...] = acc + x_ref[pl.ds(i * _BI, _BI), :]


def kernel(x, adj):
    n, f = x.shape
    grid = (n // _BI,)
    return pl.pallas_call(
        _pool_kernel,
        grid=grid,
        in_specs=[
            pl.BlockSpec((n, f), lambda i: (0, 0)),
            pl.BlockSpec((_BI, n), lambda i: (i, 0)),
        ],
        out_specs=pl.BlockSpec((_BI, f), lambda i: (i, 0)),
        out_shape=jax.ShapeDtypeStruct((n, f), jnp.float32),
        compiler_params=pltpu.CompilerParams(
            dimension_semantics=("parallel",),
            vmem_limit_bytes=120 * 1024 * 1024,
        ),
    )(x, adj)
